# baseline (device time: 367035 ns/iter reference)
import jax
import jax.numpy as jnp
from jax import lax
from jax.experimental import pallas as pl
from jax.experimental.pallas import tpu as pltpu

C = 8
NY = 3
NX = 3
NZ = C - NY - NX
N_LOCAL_CHUNKS = 16


def kernel(x):
    m, n = x.shape
    qrows = m // 4
    crows = qrows // C
    lrows = m // N_LOCAL_CHUNKS

    def body(x_ref, out_ref, vbuf, send_sems, recv_sems, in_sems, out_sems):
        my_x = lax.axis_index("x")
        my_y = lax.axis_index("y")
        my_z = lax.axis_index("z")
        y_nbr = (my_x, 1 - my_y, my_z)
        x_nbr = (1 - my_x, my_y, my_z)
        z_nbr = (my_x, my_y, 1 - my_z)

        q_me = 2 * my_x + my_z
        q_x = 2 * (1 - my_x) + my_z
        q_z = 2 * my_x + (1 - my_z)
        q_d = 2 * (1 - my_x) + (1 - my_z)

        f_base = (1 - my_y) * m

        barrier_sem = pltpu.get_barrier_semaphore()
        for nbr in (y_nbr, x_nbr, z_nbr):
            pl.semaphore_signal(
                barrier_sem, inc=1, device_id=nbr,
                device_id_type=pl.DeviceIdType.MESH,
            )
        pl.semaphore_wait(barrier_sem, 3)

        def remote(src, dst, k, dev):
            return pltpu.make_async_remote_copy(
                src_ref=src, dst_ref=dst,
                send_sem=send_sems.at[k], recv_sem=recv_sems.at[k],
                device_id=dev, device_id_type=pl.DeviceIdType.MESH,
            )

        def fq(q, c):
            return out_ref.at[pl.ds(f_base + q * qrows + c * crows, crows)]

        def own(q, c):
            return (
                x_ref.at[pl.ds(q * qrows + c * crows, crows)],
                out_ref.at[pl.ds(my_y * m + q * qrows + c * crows, crows)],
            )

        S1, S1D, S2, S3 = 0, C, C + NY, 2 * C + NY
        S4A, S4B = 3 * C + NY, 3 * C + NY + NX

        r1 = []
        for c in range(C):
            src, dst = own(q_me, c)
            r = remote(src, dst, S1 + c, y_nbr)
            r.start()
            r1.append(r)
        r1d = []
        for c in range(NY):
            src, dst = own(q_d, c)
            r = remote(src, dst, S1D + c, y_nbr)
            r.start()
            r1d.append(r)

        local_stores = []
        local_k = [0]

        def pump_local(steps):
            for _ in range(steps):
                k = local_k[0]
                if k >= N_LOCAL_CHUNKS:
                    return
                slot = k % 2
                if k >= 2:
                    local_stores[k - 2].wait()
                ld = pltpu.make_async_copy(
                    x_ref.at[pl.ds(k * lrows, lrows)], vbuf.at[slot],
                    in_sems.at[slot],
                )
                ld.start()
                ld.wait()
                st = pltpu.make_async_copy(
                    vbuf.at[slot],
                    out_ref.at[pl.ds(my_y * m + k * lrows, lrows)],
                    out_sems.at[slot],
                )
                st.start()
                local_stores.append(st)
                local_k[0] += 1

        r2, r3 = [], []
        for c in range(C):
            r1[c].wait_recv()
            r = remote(fq(q_me, c), fq(q_me, c), S2 + c, x_nbr)
            r.start()
            r2.append(r)
            r = remote(fq(q_me, c), fq(q_me, c), S3 + c, z_nbr)
            r.start()
            r3.append(r)
            pump_local(N_LOCAL_CHUNKS // C)

        r4a = []
        for i in range(NX):
            c = NY + i
            r3[c].wait_recv()
            r = remote(fq(q_z, c), fq(q_z, c), S4A + i, x_nbr)
            r.start()
            r4a.append(r)
        r4b = []
        for i in range(NZ):
            c = NY + NX + i
            r2[c].wait_recv()
            r = remote(fq(q_x, c), fq(q_x, c), S4B + i, z_nbr)
            r.start()
            r4b.append(r)

        pump_local(N_LOCAL_CHUNKS)
        for c in range(NY):
            r1d[c].wait_recv()
        for c in range(C):
            if not (NY + NX <= c < C):
                r2[c].wait_recv()
            if not (NY <= c < NY + NX):
                r3[c].wait_recv()
        for r in r4a + r4b:
            r.wait_recv()
        local_stores[-2].wait()
        local_stores[-1].wait()
        for r in r1 + r1d + r2 + r3 + r4a + r4b:
            r.wait_send()

    n_sems = 4 * C
    return pl.pallas_call(
        body,
        out_shape=jax.ShapeDtypeStruct((2 * m, n), x.dtype),
        in_specs=[pl.BlockSpec(memory_space=pl.ANY)],
        out_specs=pl.BlockSpec(memory_space=pl.ANY),
        scratch_shapes=[
            pltpu.VMEM((2, lrows, n), x.dtype),
            pltpu.SemaphoreType.DMA((n_sems,)),
            pltpu.SemaphoreType.DMA((n_sems,)),
            pltpu.SemaphoreType.DMA((2,)),
            pltpu.SemaphoreType.DMA((2,)),
        ],
        compiler_params=pltpu.CompilerParams(collective_id=0),
    )(x)


# device time: 344480 ns/iter; 1.0655x vs baseline; 1.0655x over previous
import jax
import jax.numpy as jnp
from jax import lax
from jax.experimental import pallas as pl
from jax.experimental.pallas import tpu as pltpu

C = 32
NY = 11
NX = 11
NZ = C - NY - NX
N_LOCAL_CHUNKS = 16


def kernel(x):
    m, n = x.shape
    qrows = m // 4
    crows = qrows // C
    lrows = m // N_LOCAL_CHUNKS

    def body(x_ref, out_ref, vbuf, send_sems, recv_sems, in_sems, out_sems):
        my_x = lax.axis_index("x")
        my_y = lax.axis_index("y")
        my_z = lax.axis_index("z")
        y_nbr = (my_x, 1 - my_y, my_z)
        x_nbr = (1 - my_x, my_y, my_z)
        z_nbr = (my_x, my_y, 1 - my_z)

        q_me = 2 * my_x + my_z
        q_x = 2 * (1 - my_x) + my_z
        q_z = 2 * my_x + (1 - my_z)
        q_d = 2 * (1 - my_x) + (1 - my_z)

        f_base = (1 - my_y) * m

        barrier_sem = pltpu.get_barrier_semaphore()
        for nbr in (y_nbr, x_nbr, z_nbr):
            pl.semaphore_signal(
                barrier_sem, inc=1, device_id=nbr,
                device_id_type=pl.DeviceIdType.MESH,
            )
        pl.semaphore_wait(barrier_sem, 3)

        def remote(src, dst, k, dev):
            return pltpu.make_async_remote_copy(
                src_ref=src, dst_ref=dst,
                send_sem=send_sems.at[k], recv_sem=recv_sems.at[k],
                device_id=dev, device_id_type=pl.DeviceIdType.MESH,
            )

        def fq(q, c):
            return out_ref.at[pl.ds(f_base + q * qrows + c * crows, crows)]

        def own(q, c):
            return (
                x_ref.at[pl.ds(q * qrows + c * crows, crows)],
                out_ref.at[pl.ds(my_y * m + q * qrows + c * crows, crows)],
            )

        S1, S1D, S2, S3 = 0, C, C + NY, 2 * C + NY
        S4A, S4B = 3 * C + NY, 3 * C + NY + NX

        r1 = []
        for c in range(C):
            src, dst = own(q_me, c)
            r = remote(src, dst, S1 + c, y_nbr)
            r.start()
            r1.append(r)
        r1d = []
        for c in range(NY):
            src, dst = own(q_d, c)
            r = remote(src, dst, S1D + c, y_nbr)
            r.start()
            r1d.append(r)

        local_stores = []
        local_k = [0]

        def pump_local(steps):
            for _ in range(steps):
                k = local_k[0]
                if k >= N_LOCAL_CHUNKS:
                    return
                slot = k % 2
                if k >= 2:
                    local_stores[k - 2].wait()
                ld = pltpu.make_async_copy(
                    x_ref.at[pl.ds(k * lrows, lrows)], vbuf.at[slot],
                    in_sems.at[slot],
                )
                ld.start()
                ld.wait()
                st = pltpu.make_async_copy(
                    vbuf.at[slot],
                    out_ref.at[pl.ds(my_y * m + k * lrows, lrows)],
                    out_sems.at[slot],
                )
                st.start()
                local_stores.append(st)
                local_k[0] += 1

        r2, r3 = [], []
        for c in range(C):
            r1[c].wait_recv()
            r = remote(fq(q_me, c), fq(q_me, c), S2 + c, x_nbr)
            r.start()
            r2.append(r)
            r = remote(fq(q_me, c), fq(q_me, c), S3 + c, z_nbr)
            r.start()
            r3.append(r)
            pump_local(max(1, N_LOCAL_CHUNKS // C))

        r4a = []
        for i in range(NX):
            c = NY + i
            r3[c].wait_recv()
            r = remote(fq(q_z, c), fq(q_z, c), S4A + i, x_nbr)
            r.start()
            r4a.append(r)
        r4b = []
        for i in range(NZ):
            c = NY + NX + i
            r2[c].wait_recv()
            r = remote(fq(q_x, c), fq(q_x, c), S4B + i, z_nbr)
            r.start()
            r4b.append(r)

        pump_local(N_LOCAL_CHUNKS)
        for c in range(NY):
            r1d[c].wait_recv()
        for c in range(C):
            if not (NY + NX <= c < C):
                r2[c].wait_recv()
            if not (NY <= c < NY + NX):
                r3[c].wait_recv()
        for r in r4a + r4b:
            r.wait_recv()
        local_stores[-2].wait()
        local_stores[-1].wait()
        for r in r1 + r1d + r2 + r3 + r4a + r4b:
            r.wait_send()

    n_sems = 4 * C
    return pl.pallas_call(
        body,
        out_shape=jax.ShapeDtypeStruct((2 * m, n), x.dtype),
        in_specs=[pl.BlockSpec(memory_space=pl.ANY)],
        out_specs=pl.BlockSpec(memory_space=pl.ANY),
        scratch_shapes=[
            pltpu.VMEM((2, lrows, n), x.dtype),
            pltpu.SemaphoreType.DMA((n_sems,)),
            pltpu.SemaphoreType.DMA((n_sems,)),
            pltpu.SemaphoreType.DMA((2,)),
            pltpu.SemaphoreType.DMA((2,)),
        ],
        compiler_params=pltpu.CompilerParams(collective_id=0),
    )(x)
